# baseline (speedup 1.0000x reference)
import jax
import jax.numpy as jnp
from jax.experimental import pallas as pl
from jax.experimental.pallas import tpu as pltpu

_BLOCK_ROWS = 512


def _copy_body(x_ref, o_ref):
    o_ref[...] = x_ref[...]


def kernel(x, relative_position_bias_table):
    del relative_position_bias_table
    b, s, d = x.shape
    x2 = x.reshape(b * s, d)
    rows = b * s
    out = pl.pallas_call(
        _copy_body,
        grid=(rows // _BLOCK_ROWS,),
        in_specs=[pl.BlockSpec((_BLOCK_ROWS, d), lambda i: (i, 0))],
        out_specs=pl.BlockSpec((_BLOCK_ROWS, d), lambda i: (i, 0)),
        out_shape=jax.ShapeDtypeStruct((rows, d), x.dtype),
    )(x2)
    return out.reshape(b, s, d)


# speedup vs baseline: 1.1366x; 1.1366x over previous
import jax
import jax.numpy as jnp
from jax.experimental import pallas as pl
from jax.experimental.pallas import tpu as pltpu

_BLOCK_ROWS = 2048


def _copy_body(x_ref, o_ref):
    o_ref[...] = x_ref[...]


def kernel(x, relative_position_bias_table):
    del relative_position_bias_table
    b, s, d = x.shape
    x2 = x.reshape(b * s, d)
    rows = b * s
    out = pl.pallas_call(
        _copy_body,
        grid=(rows // _BLOCK_ROWS,),
        in_specs=[pl.BlockSpec((_BLOCK_ROWS, d), lambda i: (i, 0))],
        out_specs=pl.BlockSpec((_BLOCK_ROWS, d), lambda i: (i, 0)),
        out_shape=jax.ShapeDtypeStruct((rows, d), x.dtype),
    )(x2)
    return out.reshape(b, s, d)


# speedup vs baseline: 1.1557x; 1.0168x over previous
import jax
import jax.numpy as jnp
from jax.experimental import pallas as pl
from jax.experimental.pallas import tpu as pltpu

_BR = 2048
_SLOTS = 4


def _copy_body(x_hbm, o_hbm, buf, rsem, wsem):
    rows = x_hbm.shape[0]
    chunks = rows // _BR

    def read(i):
        return pltpu.make_async_copy(
            x_hbm.at[pl.ds(i * _BR, _BR), :], buf.at[i % _SLOTS],
            rsem.at[i % _SLOTS])

    def write(i):
        return pltpu.make_async_copy(
            buf.at[i % _SLOTS], o_hbm.at[pl.ds(i * _BR, _BR), :],
            wsem.at[i % _SLOTS])

    for i in range(min(_SLOTS, chunks)):
        read(i).start()
    for i in range(chunks):
        read(i).wait()
        write(i).start()
        if i + _SLOTS < chunks:
            write(i).wait()
            read(i + _SLOTS).start()
    for i in range(max(chunks - _SLOTS, 0), chunks):
        write(i).wait()


def kernel(x, relative_position_bias_table):
    del relative_position_bias_table
    b, s, d = x.shape
    x2 = x.reshape(b * s, d)
    out = pl.pallas_call(
        _copy_body,
        in_specs=[pl.BlockSpec(memory_space=pl.ANY)],
        out_specs=pl.BlockSpec(memory_space=pl.ANY),
        out_shape=jax.ShapeDtypeStruct((b * s, d), x.dtype),
        scratch_shapes=[
            pltpu.VMEM((_SLOTS, _BR, d), x.dtype),
            pltpu.SemaphoreType.DMA((_SLOTS,)),
            pltpu.SemaphoreType.DMA((_SLOTS,)),
        ],
    )(x2)
    return out.reshape(b, s, d)
